# Initial kernel scaffold; baseline (speedup 1.0000x reference)
import jax
import jax.numpy as jnp
from jax.experimental import pallas as pl


def kernel(x, pos_edge_index, neg_edge_index, W1, b1, W2, b2, W3, b3):
    raise NotImplementedError("write your pallas kernel here")


# speedup vs baseline: 21.0300x; 21.0300x over previous
import functools

import jax
import jax.numpy as jnp
from jax import lax
from jax.experimental import pallas as pl
from jax.experimental.pallas import tpu as pltpu
from jax.experimental.pallas import tpu_sc as plsc

N = 10000
NPAD = 10240
NSCR = NPAD - N
D = 64
AUG = 80
DG = 8
NC = 2
NS = 16
NW = NC * NS
CHUNK = 128
E = 320000
E2 = 2 * E
EPW = E2 // NW
WPE = 20096
NCH = WPE // CHUNK
RPT = NPAD // NS
RPTL = N // NS

_mesh = plsc.VectorSubcoreMesh(core_axis_name="c", subcore_axis_name="s")
_sc_params = pltpu.CompilerParams(use_tc_tiling_on_sc=False,
                                  needs_layout_passes=False)


def _zero_fill(buf, nrows, width):
    zeros = jnp.zeros((16,), jnp.float32)

    def row(i, _):
        for k in range(width // 16):
            buf[i, pl.ds(k * 16, 16)] = zeros
        return 0

    lax.fori_loop(0, nrows, row, 0)


def _round_body(s_hbm, src_hbm, dst_hbm, out_hbm,
                idx_s, idx_d, rowbuf, zbuf, acc):
    cid = lax.axis_index("c")
    sid = lax.axis_index("s")
    wid = sid * NC + cid
    r0 = sid * RPT

    _zero_fill(zbuf, CHUNK, AUG)
    for k in range(RPT // CHUNK):
        pltpu.sync_copy(zbuf, acc.at[pl.ds(r0 + k * CHUNK, CHUNK)])
    pltpu.sync_copy(src_hbm.at[wid], idx_s)
    pltpu.sync_copy(dst_hbm.at[wid], idx_d)
    plsc.subcore_barrier()

    def chunk(j, _):
        pltpu.sync_copy(s_hbm.at[idx_s.at[j]], rowbuf)
        pltpu.sync_copy(rowbuf, acc.at[idx_d.at[j]], add=True)
        return 0

    lax.fori_loop(0, NCH, chunk, 0)
    plsc.subcore_barrier()
    pltpu.sync_copy(acc.at[pl.ds(r0, RPT)], out_hbm.at[cid, pl.ds(r0, RPT)])


_round = pl.kernel(
    _round_body,
    out_type=jax.ShapeDtypeStruct((NC, NPAD, AUG), jnp.float32),
    mesh=_mesh,
    compiler_params=_sc_params,
    scratch_types=[
        pltpu.VMEM((NCH, CHUNK), jnp.int32),
        pltpu.VMEM((NCH, CHUNK), jnp.int32),
        pltpu.VMEM((CHUNK, AUG), jnp.float32),
        pltpu.VMEM((CHUNK, AUG), jnp.float32),
        pltpu.VMEM_SHARED((NPAD, AUG), jnp.float32),
    ],
)


def _deg_body(dst_hbm, cst_hbm, out_hbm, idx_d, ubuf, acc):
    cid = lax.axis_index("c")
    sid = lax.axis_index("s")
    wid = sid * NC + cid
    r0 = sid * RPT

    pltpu.sync_copy(cst_hbm, ubuf)
    for k in range(RPT // CHUNK):
        pltpu.sync_copy(ubuf.at[pl.ds(CHUNK, CHUNK)],
                        acc.at[pl.ds(r0 + k * CHUNK, CHUNK)])
    pltpu.sync_copy(dst_hbm.at[wid], idx_d)
    plsc.subcore_barrier()

    def chunk(j, _):
        pltpu.sync_copy(ubuf.at[pl.ds(0, CHUNK)], acc.at[idx_d.at[j]],
                        add=True)
        return 0

    lax.fori_loop(0, NCH, chunk, 0)
    plsc.subcore_barrier()
    pltpu.sync_copy(acc.at[pl.ds(r0, RPT)], out_hbm.at[cid, pl.ds(r0, RPT)])


_deg_round = pl.kernel(
    _deg_body,
    out_type=jax.ShapeDtypeStruct((NC, NPAD, DG), jnp.float32),
    mesh=_mesh,
    compiler_params=_sc_params,
    scratch_types=[
        pltpu.VMEM((NCH, CHUNK), jnp.int32),
        pltpu.VMEM((2 * CHUNK, DG), jnp.float32),
        pltpu.VMEM_SHARED((NPAD, DG), jnp.float32),
    ],
)


def _logits_body(z_hbm, i0_hbm, i1_hbm, out_hbm,
                 idx0, idx1, rowa, rowb, lbuf, fbuf, table):
    cid = lax.axis_index("c")
    sid = lax.axis_index("s")
    wid = sid * NC + cid
    r0 = sid * RPTL

    pltpu.sync_copy(z_hbm.at[pl.ds(r0, RPTL)], table.at[pl.ds(r0, RPTL)])
    pltpu.sync_copy(i0_hbm.at[wid], idx0)
    pltpu.sync_copy(i1_hbm.at[wid], idx1)
    plsc.subcore_barrier()

    col = lax.iota(jnp.int32, 16) * 16

    def chunk(j, _):
        pltpu.sync_copy(table.at[idx0.at[j]], rowa)
        pltpu.sync_copy(table.at[idx1.at[j]], rowb)

        def group(g, _):
            for e16 in range(16):
                e = g * 16 + e16
                c = rowa[e, pl.ds(0, 16)] * rowb[e, pl.ds(0, 16)]
                for k in range(1, D // 16):
                    c += rowa[e, pl.ds(k * 16, 16)] * rowb[e, pl.ds(k * 16, 16)]
                fbuf[pl.ds(e16 * 16, 16)] = c
            acc = plsc.load_gather(fbuf, [col])
            for jj in range(1, 16):
                acc = acc + plsc.load_gather(fbuf, [col + jj])
            lbuf[pl.ds(j * CHUNK + g * 16, 16)] = acc
            return 0

        lax.fori_loop(0, CHUNK // 16, group, 0)
        return 0

    lax.fori_loop(0, NCH, chunk, 0)
    pltpu.sync_copy(lbuf, out_hbm.at[pl.ds(wid * WPE, WPE)])


_logits_call = pl.kernel(
    _logits_body,
    out_type=jax.ShapeDtypeStruct((NW * WPE,), jnp.float32),
    mesh=_mesh,
    compiler_params=_sc_params,
    scratch_types=[
        pltpu.VMEM((NCH, CHUNK), jnp.int32),
        pltpu.VMEM((NCH, CHUNK), jnp.int32),
        pltpu.VMEM((CHUNK, D), jnp.float32),
        pltpu.VMEM((CHUNK, D), jnp.float32),
        pltpu.VMEM((WPE,), jnp.float32),
        pltpu.VMEM((256,), jnp.float32),
        pltpu.VMEM_SHARED((N, D), jnp.float32),
    ],
)


def _tc_prep_body(x_ref, w1_ref, w2_ref, w3_ref, b1_ref, b2_ref,
                  xc_ref, bc1_ref, bc2_ref):
    w23 = jnp.dot(w2_ref[...], w3_ref[...], preferred_element_type=jnp.float32)
    wc = jnp.dot(w1_ref[...], w23, preferred_element_type=jnp.float32)
    xc_ref[...] = jnp.dot(x_ref[...], wc, preferred_element_type=jnp.float32)
    bc1_ref[...] = jnp.dot(b1_ref[...], w23, preferred_element_type=jnp.float32)
    bc2_ref[...] = jnp.dot(b2_ref[...], w3_ref[...],
                           preferred_element_type=jnp.float32)


def _tc_prep(x_pad, W1, W2, W3, b1, b2):
    return pl.pallas_call(
        _tc_prep_body,
        out_shape=[
            jax.ShapeDtypeStruct((NPAD, D), jnp.float32),
            jax.ShapeDtypeStruct((1, D), jnp.float32),
            jax.ShapeDtypeStruct((1, D), jnp.float32),
        ],
    )(x_pad, W1, W2, W3, b1, b2)


def _pad_plan(idx, mod):
    body = idx.reshape(NW, EPW)
    npad = WPE - EPW
    base = N if mod == NSCR else 0
    padv = (base + (jnp.arange(NW * npad, dtype=jnp.int32) % mod)
            ).reshape(NW, npad)
    return jnp.concatenate([body, padv], axis=1).reshape(NW, NCH, CHUNK)


def kernel(x, pos_edge_index, neg_edge_index, W1, b1, W2, b2, W3, b3):
    pos = pos_edge_index.astype(jnp.int32)
    neg = neg_edge_index.astype(jnp.int32)
    src = _pad_plan(jnp.concatenate([pos[0], pos[1]]), NSCR)
    dst = _pad_plan(jnp.concatenate([pos[1], pos[0]]), NSCR)
    sel0 = _pad_plan(jnp.concatenate([pos[0], neg[0]]), N)
    sel1 = _pad_plan(jnp.concatenate([pos[1], neg[1]]), N)

    cst = jnp.zeros((2 * CHUNK, DG), jnp.float32).at[:CHUNK, 0].set(1.0)
    dp = _deg_round(dst, cst)
    deg = dp[0, :, 0] + dp[1, :, 0] + 1.0
    real = jnp.arange(NPAD) < N
    dinv = jnp.where(real, lax.rsqrt(deg), 0.0)
    dinv2 = dinv * dinv

    x_pad = jnp.pad(x, ((0, NPAD - N), (0, 0)))
    xc, bc1, bc2 = _tc_prep(x_pad, W1, W2, W3, b1[None, :], b2[None, :])

    aug = jnp.concatenate(
        [xc, jnp.ones((NPAD, 1), jnp.float32),
         jnp.zeros((NPAD, AUG - D - 1), jnp.float32)], axis=1)
    s = dinv[:, None] * aug
    p = _round(s, src, dst)
    tot = p[0] + p[1] + s
    u1 = dinv * tot[:, D]
    s = dinv2[:, None] * tot
    p = _round(s, src, dst)
    tot = p[0] + p[1] + s
    u2 = dinv * tot[:, D]
    s = dinv2[:, None] * tot
    p = _round(s, src, dst)
    tot = p[0] + p[1] + s

    z = dinv[:, None] * tot[:, :D] + u2[:, None] * bc1[0] \
        + u1[:, None] * bc2[0] + b3[None, :]

    lp = _logits_call(z[:N], sel0, sel1)
    return lp.reshape(NW, WPE)[:, :EPW].reshape(-1)
